# baseline (speedup 1.0000x reference)
import functools

import jax
import jax.numpy as jnp
from jax import lax
from jax.experimental import pallas as pl
from jax.experimental.pallas import tpu as pltpu
from jax.experimental.pallas import tpu_sc as plsc

N_NODES = 100000
N_EDGES = 1600000
H_DIM = 128
NUM_GRAPHS = 128
BN_EPS = 1e-5

N_PAD = 102400
ROWS = 5000
NB = N_NODES // ROWS


def _mm_stats_body(u_ref, w_ref, b_ref, z_ref, stats_ref, acc_ref):
    i = pl.program_id(0)

    @pl.when(i == 0)
    def _():
        acc_ref[...] = jnp.zeros_like(acc_ref)

    z = jnp.dot(u_ref[...], w_ref[...], preferred_element_type=jnp.float32)
    z = z + b_ref[0, :][None, :]
    z_ref[...] = z
    s = jnp.sum(z, axis=0)
    sq = jnp.sum(z * z, axis=0)
    acc_ref[0, :] += s
    acc_ref[1, :] += sq

    @pl.when(i == NB - 1)
    def _():
        stats_ref[...] = acc_ref[...]


def _mm_stats(u, W, b):
    d_in = u.shape[1]
    return pl.pallas_call(
        _mm_stats_body,
        grid=(NB,),
        in_specs=[
            pl.BlockSpec((ROWS, d_in), lambda i: (i, 0)),
            pl.BlockSpec((d_in, H_DIM), lambda i: (0, 0)),
            pl.BlockSpec((1, H_DIM), lambda i: (0, 0)),
        ],
        out_specs=[
            pl.BlockSpec((ROWS, H_DIM), lambda i: (i, 0)),
            pl.BlockSpec((2, H_DIM), lambda i: (0, 0)),
        ],
        out_shape=[
            jax.ShapeDtypeStruct((N_PAD, H_DIM), jnp.float32),
            jax.ShapeDtypeStruct((2, H_DIM), jnp.float32),
        ],
        scratch_shapes=[pltpu.VMEM((2, H_DIM), jnp.float32)],
    )(u, W, b.reshape(1, H_DIM))


def _mm_plain_body(x_ref, w_ref, y_ref):
    y_ref[...] = jnp.dot(x_ref[...], w_ref[...],
                         preferred_element_type=jnp.float32)


def _mm_plain(x, W):
    d_in = x.shape[1]
    return pl.pallas_call(
        _mm_plain_body,
        grid=(NB,),
        in_specs=[
            pl.BlockSpec((ROWS, d_in), lambda i: (i, 0)),
            pl.BlockSpec((d_in, H_DIM), lambda i: (0, 0)),
        ],
        out_specs=pl.BlockSpec((ROWS, H_DIM), lambda i: (i, 0)),
        out_shape=jax.ShapeDtypeStruct((N_PAD, H_DIM), jnp.float32),
    )(x, W)


def _stats_body(u_ref, stats_ref, acc_ref):
    i = pl.program_id(0)

    @pl.when(i == 0)
    def _():
        acc_ref[...] = jnp.zeros_like(acc_ref)

    u = u_ref[...]
    acc_ref[0, :] += jnp.sum(u, axis=0)
    acc_ref[1, :] += jnp.sum(u * u, axis=0)

    @pl.when(i == NB - 1)
    def _():
        stats_ref[...] = acc_ref[...]


def _stats(u):
    return pl.pallas_call(
        _stats_body,
        grid=(NB,),
        in_specs=[pl.BlockSpec((ROWS, H_DIM), lambda i: (i, 0))],
        out_specs=pl.BlockSpec((2, H_DIM), lambda i: (0, 0)),
        out_shape=jax.ShapeDtypeStruct((2, H_DIM), jnp.float32),
        scratch_shapes=[pltpu.VMEM((2, H_DIM), jnp.float32)],
    )(u)


def _norm_body(z_ref, stats_ref, g_ref, bt_ref, h_ref):
    mean = stats_ref[0, :] * (1.0 / N_NODES)
    var = stats_ref[1, :] * (1.0 / N_NODES) - mean * mean
    scale = g_ref[0, :] * lax.rsqrt(var + BN_EPS)
    shift = bt_ref[0, :] - mean * scale
    h_ref[...] = jnp.maximum(z_ref[...] * scale[None, :] + shift[None, :], 0.0)


def _norm(z, stats, g, bt):
    return pl.pallas_call(
        _norm_body,
        grid=(NB,),
        in_specs=[
            pl.BlockSpec((ROWS, H_DIM), lambda i: (i, 0)),
            pl.BlockSpec((2, H_DIM), lambda i: (0, 0)),
            pl.BlockSpec((1, H_DIM), lambda i: (0, 0)),
            pl.BlockSpec((1, H_DIM), lambda i: (0, 0)),
        ],
        out_specs=pl.BlockSpec((ROWS, H_DIM), lambda i: (i, 0)),
        out_shape=jax.ShapeDtypeStruct((N_PAD, H_DIM), jnp.float32),
    )(z, stats, g.reshape(1, H_DIM), bt.reshape(1, H_DIM))


def _normpool_body(z_ref, stats_ref, g_ref, bt_ref, batch_ref, wfc_ref, bfc_ref,
                   out_ref, pool_ref, cnt_ref):
    i = pl.program_id(0)

    @pl.when(i == 0)
    def _():
        pool_ref[...] = jnp.zeros_like(pool_ref)
        cnt_ref[...] = jnp.zeros_like(cnt_ref)

    mean = stats_ref[0, :] * (1.0 / N_NODES)
    var = stats_ref[1, :] * (1.0 / N_NODES) - mean * mean
    scale = g_ref[0, :] * lax.rsqrt(var + BN_EPS)
    shift = bt_ref[0, :] - mean * scale
    h = jnp.maximum(z_ref[...] * scale[None, :] + shift[None, :], 0.0)

    bb = batch_ref[0, 0, :]
    onehot = (bb[:, None] == lax.broadcasted_iota(jnp.int32, (ROWS, NUM_GRAPHS), 1)
              ).astype(jnp.float32)
    pool_ref[...] += lax.dot_general(onehot, h, (((0,), (0,)), ((), ())),
                                     preferred_element_type=jnp.float32)
    cnt_ref[0, :] += jnp.sum(onehot, axis=0)

    @pl.when(i == NB - 1)
    def _():
        counts = jnp.maximum(cnt_ref[0, :], 1.0)
        pooled = pool_ref[...] / counts[:, None]
        logit = jnp.dot(pooled, wfc_ref[...],
                        preferred_element_type=jnp.float32) + bfc_ref[0, 0]
        out_ref[...] = 1.0 / (1.0 + jnp.exp(-logit))


def _normpool(z, stats, g, bt, batch, Wfc, bfc):
    batch3 = batch.reshape(NB, 1, ROWS)
    return pl.pallas_call(
        _normpool_body,
        grid=(NB,),
        in_specs=[
            pl.BlockSpec((ROWS, H_DIM), lambda i: (i, 0)),
            pl.BlockSpec((2, H_DIM), lambda i: (0, 0)),
            pl.BlockSpec((1, H_DIM), lambda i: (0, 0)),
            pl.BlockSpec((1, H_DIM), lambda i: (0, 0)),
            pl.BlockSpec((1, 1, ROWS), lambda i: (i, 0, 0)),
            pl.BlockSpec((H_DIM, 1), lambda i: (0, 0)),
            pl.BlockSpec((1, 1), lambda i: (0, 0)),
        ],
        out_specs=pl.BlockSpec((NUM_GRAPHS, 1), lambda i: (0, 0)),
        out_shape=jax.ShapeDtypeStruct((NUM_GRAPHS, 1), jnp.float32),
        scratch_shapes=[
            pltpu.VMEM((NUM_GRAPHS, H_DIM), jnp.float32),
            pltpu.VMEM((1, NUM_GRAPHS), jnp.float32),
        ],
    )(z, stats, g.reshape(1, H_DIM), bt.reshape(1, H_DIM), batch3, Wfc,
      bfc.reshape(1, 1))


_NTILES = 16
_NCORES = 2
_LANES = 16
_EB = 2000
_NVR = _EB // _LANES
_PER_TILE_E = N_EDGES // _NTILES
_NEB = _PER_TILE_E // _EB
_B = 64
_NQ = 3
_GRP = _NQ * _B
_C = 12800
_NCH = (N_PAD // _C) // _NCORES
_CAPB = 14808
_UNROLL = 5
_SRC_BITS = 17


def _sc_bucket_build():
    def body(src_hbm, dst_hbm, pk_hbm, cnts_hbm, srcblk, dstblk, stg_pk,
             cnts_v):
        c = lax.axis_index("c")
        s = lax.axis_index("s")
        ebase = s * _PER_TILE_E
        lane = lax.iota(jnp.int32, _LANES)
        cnts_vec = lane * 0

        for ci in range(_NCH):
            chunk = ci * _NCORES + c
            lo = chunk * _C

            def eb_body(eb, cnt):
                pltpu.sync_copy(src_hbm.at[pl.ds(ebase + eb * _EB, _EB)],
                                srcblk)
                pltpu.sync_copy(dst_hbm.at[pl.ds(ebase + eb * _EB, _EB)],
                                dstblk)

                def k_body(k, cnt):
                    for grp in ((0, 1), (2, 3), (4,)):
                        cums, pks, ms = [], [], []
                        for u in grp:
                            off = (k * _UNROLL + u) * _LANES
                            sv = srcblk[pl.ds(off, _LANES)]
                            dv = dstblk[pl.ds(off, _LANES)]
                            m = (dv >= lo) & (dv < lo + _C)
                            cums.append(plsc.cumsum(m.astype(jnp.int32)))
                            pks.append(sv + lax.shift_left(
                                dv - lo, jnp.int32(_SRC_BITS)))
                            ms.append(m)
                        for i in range(len(grp)):
                            pos = cnt + cums[i] - 1
                            plsc.store_scatter(stg_pk, [pos], pks[i],
                                               mask=ms[i])
                            cnt = cnt + cums[i][_LANES - 1]
                    return cnt

                return lax.fori_loop(0, _NVR // _UNROLL, k_body, cnt)

            cnt = lax.fori_loop(0, _NEB, eb_body, jnp.int32(0))

            pad = (lane + jnp.int32(_LANES)) + lax.shift_left(
                jnp.int32(_C) + (lane & 7), jnp.int32(_SRC_BITS))
            for i in range(_GRP // _LANES):
                stg_pk[pl.ds(cnt + i * _LANES, _LANES)] = pad + jnp.int32(i)

            region = (((c * _NCH) + ci) * _NTILES + s) * _CAPB
            pltpu.sync_copy(stg_pk, pk_hbm.at[pl.ds(region, _CAPB)])
            cnts_vec = jnp.where(lane == ci, cnt, cnts_vec)

        cnts_v[...] = cnts_vec
        pltpu.sync_copy(cnts_v,
                        cnts_hbm.at[pl.ds((c * _NTILES + s) * _LANES, _LANES)])

    return pl.kernel(
        body,
        mesh=plsc.VectorSubcoreMesh(core_axis_name="c", subcore_axis_name="s"),
        compiler_params=pltpu.CompilerParams(needs_layout_passes=False),
        out_type=[
            jax.ShapeDtypeStruct((_NCORES * _NCH * _NTILES * _CAPB,),
                                 jnp.int32),
            jax.ShapeDtypeStruct((_NCORES * _NTILES * _LANES,), jnp.int32),
        ],
        scratch_types=[
            pltpu.VMEM((_EB,), jnp.int32),
            pltpu.VMEM((_EB,), jnp.int32),
            pltpu.VMEM((_CAPB,), jnp.int32),
            pltpu.VMEM((_LANES,), jnp.int32),
        ],
    )


_sc_bucket = _sc_bucket_build()


def _sc_aggregate_build(D):
    PR = _C // _NTILES

    def body(h_hbm, pk_hbm, cnts_hbm, u_hbm,
             pkblk, cnts_v, ex_src, ex_dst, rows, sem_g, sem_s, u_sh):
        c = lax.axis_index("c")
        s = lax.axis_index("s")
        mask_src = jnp.int32((1 << _SRC_BITS) - 1)

        pltpu.sync_copy(cnts_hbm.at[pl.ds((c * _NTILES + s) * _LANES, _LANES)],
                        cnts_v)
        cv = cnts_v[...]

        for ci in range(_NCH):
            chunk = ci * _NCORES + c
            lo = chunk * _C

            pltpu.sync_copy(h_hbm.at[pl.ds(lo + s * PR, PR)],
                            u_sh.at[pl.ds(s * PR, PR)])
            plsc.subcore_barrier()

            cnt = cv[ci]
            region = (((c * _NCH) + ci) * _NTILES + s) * _CAPB

            def group(g, carry):
                pltpu.sync_copy(pk_hbm.at[pl.ds(region + g * _GRP, _GRP)],
                                pkblk)
                for q in range(_NQ):
                    for r in range(_B // _LANES):
                        pk = pkblk[pl.ds(q * _B + r * _LANES, _LANES)]
                        ex_src[q, pl.ds(r * _LANES, _LANES)] = pk & mask_src
                        ex_dst[q, pl.ds(r * _LANES, _LANES)] = \
                            lax.shift_right_logical(pk, jnp.int32(_SRC_BITS))
                for q in range(_NQ):
                    pltpu.async_copy(h_hbm.at[ex_src.at[q]], rows.at[q], sem_g)
                for q in range(_NQ):
                    pltpu.make_async_copy(h_hbm.at[ex_src.at[q]], rows.at[q],
                                          sem_g).wait()
                for q in range(_NQ):
                    pltpu.async_copy(rows.at[q], u_sh.at[ex_dst.at[q]], sem_s,
                                     add=True)
                for q in range(_NQ):
                    pltpu.make_async_copy(rows.at[q], u_sh.at[ex_dst.at[q]],
                                          sem_s).wait()
                return carry

            lax.fori_loop(0, cnt // _GRP + 1, group, jnp.int32(0))
            plsc.subcore_barrier()

            pltpu.sync_copy(u_sh.at[pl.ds(s * PR, PR)],
                            u_hbm.at[pl.ds(lo + s * PR, PR)])
            plsc.subcore_barrier()

    return pl.kernel(
        body,
        mesh=plsc.VectorSubcoreMesh(core_axis_name="c", subcore_axis_name="s"),
        compiler_params=pltpu.CompilerParams(needs_layout_passes=False),
        out_type=jax.ShapeDtypeStruct((N_PAD, D), jnp.float32),
        scratch_types=[
            pltpu.VMEM((_GRP,), jnp.int32),
            pltpu.VMEM((_LANES,), jnp.int32),
            pltpu.VMEM((_NQ, _B), jnp.int32),
            pltpu.VMEM((_NQ, _B), jnp.int32),
            pltpu.VMEM((_NQ, _B, D), jnp.float32),
            pltpu.SemaphoreType.DMA,
            pltpu.SemaphoreType.DMA,
            pltpu.VMEM_SHARED((_C + 8, D), jnp.float32),
        ],
    )


_sc_agg_128 = _sc_aggregate_build(H_DIM)


def _aggregate(h, pk, cnts):
    return _sc_agg_128(h, pk, cnts)


def kernel(x, edge_index, batch, W1, b1, g1, bt1, W2, b2, g2, bt2,
           W3, b3, g3, bt3, Wfc, bfc):
    src = edge_index[0].astype(jnp.int32)
    dst = edge_index[1].astype(jnp.int32)
    batch = batch.astype(jnp.int32)

    pk, cnts = _sc_bucket(src, dst)
    y1 = _mm_plain(x, W1)
    u1 = _aggregate(y1, pk, cnts)
    s1 = _stats(u1)
    h1 = _norm(u1, s1, g1, bt1)

    u2 = _aggregate(h1, pk, cnts)
    z2, s2 = _mm_stats(u2, W2, b2)
    h2 = _norm(z2, s2, g2, bt2)

    u3 = _aggregate(h2, pk, cnts)
    z3, s3 = _mm_stats(u3, W3, b3)
    return _normpool(z3, s3, g3, bt3, batch, Wfc, bfc)


# speedup vs baseline: 1.2460x; 1.2460x over previous
import functools

import jax
import jax.numpy as jnp
from jax import lax
from jax.experimental import pallas as pl
from jax.experimental.pallas import tpu as pltpu
from jax.experimental.pallas import tpu_sc as plsc

N_NODES = 100000
N_EDGES = 1600000
H_DIM = 128
NUM_GRAPHS = 128
BN_EPS = 1e-5

N_PAD = 102400
ROWS = 5000
NB = N_NODES // ROWS


def _mm_stats_body(u_ref, w_ref, b_ref, z_ref, stats_ref, acc_ref):
    i = pl.program_id(0)

    @pl.when(i == 0)
    def _():
        acc_ref[...] = jnp.zeros_like(acc_ref)

    z = jnp.dot(u_ref[...], w_ref[...], preferred_element_type=jnp.float32)
    z = z + b_ref[0, :][None, :]
    z_ref[...] = z
    s = jnp.sum(z, axis=0)
    sq = jnp.sum(z * z, axis=0)
    acc_ref[0, :] += s
    acc_ref[1, :] += sq

    @pl.when(i == NB - 1)
    def _():
        stats_ref[...] = acc_ref[...]


def _mm_stats(u, W, b):
    d_in = u.shape[1]
    return pl.pallas_call(
        _mm_stats_body,
        grid=(NB,),
        in_specs=[
            pl.BlockSpec((ROWS, d_in), lambda i: (i, 0)),
            pl.BlockSpec((d_in, H_DIM), lambda i: (0, 0)),
            pl.BlockSpec((1, H_DIM), lambda i: (0, 0)),
        ],
        out_specs=[
            pl.BlockSpec((ROWS, H_DIM), lambda i: (i, 0)),
            pl.BlockSpec((2, H_DIM), lambda i: (0, 0)),
        ],
        out_shape=[
            jax.ShapeDtypeStruct((N_PAD, H_DIM), jnp.float32),
            jax.ShapeDtypeStruct((2, H_DIM), jnp.float32),
        ],
        scratch_shapes=[pltpu.VMEM((2, H_DIM), jnp.float32)],
    )(u, W, b.reshape(1, H_DIM))


def _mm_plain_body(x_ref, w_ref, y_ref):
    y_ref[...] = jnp.dot(x_ref[...], w_ref[...],
                         preferred_element_type=jnp.float32)


def _mm_plain(x, W):
    d_in = x.shape[1]
    return pl.pallas_call(
        _mm_plain_body,
        grid=(NB,),
        in_specs=[
            pl.BlockSpec((ROWS, d_in), lambda i: (i, 0)),
            pl.BlockSpec((d_in, H_DIM), lambda i: (0, 0)),
        ],
        out_specs=pl.BlockSpec((ROWS, H_DIM), lambda i: (i, 0)),
        out_shape=jax.ShapeDtypeStruct((N_PAD, H_DIM), jnp.float32),
    )(x, W)


def _stats_body(u_ref, stats_ref, acc_ref):
    i = pl.program_id(0)

    @pl.when(i == 0)
    def _():
        acc_ref[...] = jnp.zeros_like(acc_ref)

    u = u_ref[...]
    acc_ref[0, :] += jnp.sum(u, axis=0)
    acc_ref[1, :] += jnp.sum(u * u, axis=0)

    @pl.when(i == NB - 1)
    def _():
        stats_ref[...] = acc_ref[...]


def _stats(u):
    return pl.pallas_call(
        _stats_body,
        grid=(NB,),
        in_specs=[pl.BlockSpec((ROWS, H_DIM), lambda i: (i, 0))],
        out_specs=pl.BlockSpec((2, H_DIM), lambda i: (0, 0)),
        out_shape=jax.ShapeDtypeStruct((2, H_DIM), jnp.float32),
        scratch_shapes=[pltpu.VMEM((2, H_DIM), jnp.float32)],
    )(u)


def _norm_body(z_ref, stats_ref, g_ref, bt_ref, h_ref):
    mean = stats_ref[0, :] * (1.0 / N_NODES)
    var = stats_ref[1, :] * (1.0 / N_NODES) - mean * mean
    scale = g_ref[0, :] * lax.rsqrt(var + BN_EPS)
    shift = bt_ref[0, :] - mean * scale
    h_ref[...] = jnp.maximum(z_ref[...] * scale[None, :] + shift[None, :], 0.0)


def _norm(z, stats, g, bt):
    return pl.pallas_call(
        _norm_body,
        grid=(NB,),
        in_specs=[
            pl.BlockSpec((ROWS, H_DIM), lambda i: (i, 0)),
            pl.BlockSpec((2, H_DIM), lambda i: (0, 0)),
            pl.BlockSpec((1, H_DIM), lambda i: (0, 0)),
            pl.BlockSpec((1, H_DIM), lambda i: (0, 0)),
        ],
        out_specs=pl.BlockSpec((ROWS, H_DIM), lambda i: (i, 0)),
        out_shape=jax.ShapeDtypeStruct((N_PAD, H_DIM), jnp.float32),
    )(z, stats, g.reshape(1, H_DIM), bt.reshape(1, H_DIM))


def _normpool_body(z_ref, stats_ref, g_ref, bt_ref, batch_ref, wfc_ref, bfc_ref,
                   out_ref, pool_ref, cnt_ref):
    i = pl.program_id(0)

    @pl.when(i == 0)
    def _():
        pool_ref[...] = jnp.zeros_like(pool_ref)
        cnt_ref[...] = jnp.zeros_like(cnt_ref)

    mean = stats_ref[0, :] * (1.0 / N_NODES)
    var = stats_ref[1, :] * (1.0 / N_NODES) - mean * mean
    scale = g_ref[0, :] * lax.rsqrt(var + BN_EPS)
    shift = bt_ref[0, :] - mean * scale
    h = jnp.maximum(z_ref[...] * scale[None, :] + shift[None, :], 0.0)

    bb = batch_ref[0, 0, :]
    onehot = (bb[:, None] == lax.broadcasted_iota(jnp.int32, (ROWS, NUM_GRAPHS), 1)
              ).astype(jnp.float32)
    pool_ref[...] += lax.dot_general(onehot, h, (((0,), (0,)), ((), ())),
                                     preferred_element_type=jnp.float32)
    cnt_ref[0, :] += jnp.sum(onehot, axis=0)

    @pl.when(i == NB - 1)
    def _():
        counts = jnp.maximum(cnt_ref[0, :], 1.0)
        pooled = pool_ref[...] / counts[:, None]
        logit = jnp.dot(pooled, wfc_ref[...],
                        preferred_element_type=jnp.float32) + bfc_ref[0, 0]
        out_ref[...] = 1.0 / (1.0 + jnp.exp(-logit))


def _normpool(z, stats, g, bt, batch, Wfc, bfc):
    batch3 = batch.reshape(NB, 1, ROWS)
    return pl.pallas_call(
        _normpool_body,
        grid=(NB,),
        in_specs=[
            pl.BlockSpec((ROWS, H_DIM), lambda i: (i, 0)),
            pl.BlockSpec((2, H_DIM), lambda i: (0, 0)),
            pl.BlockSpec((1, H_DIM), lambda i: (0, 0)),
            pl.BlockSpec((1, H_DIM), lambda i: (0, 0)),
            pl.BlockSpec((1, 1, ROWS), lambda i: (i, 0, 0)),
            pl.BlockSpec((H_DIM, 1), lambda i: (0, 0)),
            pl.BlockSpec((1, 1), lambda i: (0, 0)),
        ],
        out_specs=pl.BlockSpec((NUM_GRAPHS, 1), lambda i: (0, 0)),
        out_shape=jax.ShapeDtypeStruct((NUM_GRAPHS, 1), jnp.float32),
        scratch_shapes=[
            pltpu.VMEM((NUM_GRAPHS, H_DIM), jnp.float32),
            pltpu.VMEM((1, NUM_GRAPHS), jnp.float32),
        ],
    )(z, stats, g.reshape(1, H_DIM), bt.reshape(1, H_DIM), batch3, Wfc,
      bfc.reshape(1, 1))


_NTILES = 16
_NCORES = 2
_LANES = 16
_EB = 2000
_NVR = _EB // _LANES
_PER_TILE_E = N_EDGES // _NTILES
_NEB = _PER_TILE_E // _EB
_B = 64
_NQ = 3
_GRP = _NQ * _B
_C = 12800
_NCH = (N_PAD // _C) // _NCORES
_CAPB = 14808
_UNROLL = 5
_SRC_BITS = 17


def _sc_bucket_build():
    def body(src_hbm, dst_hbm, pk_hbm, cnts_hbm, srcblk, dstblk, stg_pk,
             cnts_v):
        c = lax.axis_index("c")
        s = lax.axis_index("s")
        ebase = s * _PER_TILE_E
        lane = lax.iota(jnp.int32, _LANES)
        cnts_vec = lane * 0

        for ci in range(_NCH):
            chunk = ci * _NCORES + c
            lo = chunk * _C

            def eb_body(eb, cnt):
                pltpu.sync_copy(src_hbm.at[pl.ds(ebase + eb * _EB, _EB)],
                                srcblk)
                pltpu.sync_copy(dst_hbm.at[pl.ds(ebase + eb * _EB, _EB)],
                                dstblk)

                def k_body(k, cnt):
                    for grp in ((0, 1), (2, 3), (4,)):
                        cums, pks, ms = [], [], []
                        for u in grp:
                            off = (k * _UNROLL + u) * _LANES
                            sv = srcblk[pl.ds(off, _LANES)]
                            dv = dstblk[pl.ds(off, _LANES)]
                            m = (dv >= lo) & (dv < lo + _C)
                            cums.append(plsc.cumsum(m.astype(jnp.int32)))
                            pks.append(sv + lax.shift_left(
                                dv - lo, jnp.int32(_SRC_BITS)))
                            ms.append(m)
                        for i in range(len(grp)):
                            pos = cnt + cums[i] - 1
                            plsc.store_scatter(stg_pk, [pos], pks[i],
                                               mask=ms[i])
                            cnt = cnt + cums[i][_LANES - 1]
                    return cnt

                return lax.fori_loop(0, _NVR // _UNROLL, k_body, cnt)

            cnt = lax.fori_loop(0, _NEB, eb_body, jnp.int32(0))

            pad = (lane + jnp.int32(_LANES)) + lax.shift_left(
                jnp.int32(_C) + (lane & 7), jnp.int32(_SRC_BITS))
            for i in range(_GRP // _LANES):
                stg_pk[pl.ds(cnt + i * _LANES, _LANES)] = pad + jnp.int32(i)

            region = (((c * _NCH) + ci) * _NTILES + s) * _CAPB
            pltpu.sync_copy(stg_pk, pk_hbm.at[pl.ds(region, _CAPB)])
            cnts_vec = jnp.where(lane == ci, cnt, cnts_vec)

        cnts_v[...] = cnts_vec
        pltpu.sync_copy(cnts_v,
                        cnts_hbm.at[pl.ds((c * _NTILES + s) * _LANES, _LANES)])

    return pl.kernel(
        body,
        mesh=plsc.VectorSubcoreMesh(core_axis_name="c", subcore_axis_name="s"),
        compiler_params=pltpu.CompilerParams(needs_layout_passes=False),
        out_type=[
            jax.ShapeDtypeStruct((_NCORES * _NCH * _NTILES * _CAPB,),
                                 jnp.int32),
            jax.ShapeDtypeStruct((_NCORES * _NTILES * _LANES,), jnp.int32),
        ],
        scratch_types=[
            pltpu.VMEM((_EB,), jnp.int32),
            pltpu.VMEM((_EB,), jnp.int32),
            pltpu.VMEM((_CAPB,), jnp.int32),
            pltpu.VMEM((_LANES,), jnp.int32),
        ],
    )


_sc_bucket = _sc_bucket_build()


def _sc_aggregate_build(D):
    PR = _C // _NTILES

    def body(h_hbm, pk_hbm, cnts_hbm, u_hbm,
             pkblk, cnts_v, ex_src, ex_dst, rows, sem_g, sem_s, sem_pk, u_sh):
        c = lax.axis_index("c")
        s = lax.axis_index("s")
        mask_src = jnp.int32((1 << _SRC_BITS) - 1)

        pltpu.sync_copy(cnts_hbm.at[pl.ds((c * _NTILES + s) * _LANES, _LANES)],
                        cnts_v)
        cv = cnts_v[...]

        for ci in range(_NCH):
            chunk = ci * _NCORES + c
            lo = chunk * _C

            pltpu.sync_copy(h_hbm.at[pl.ds(lo + s * PR, PR)],
                            u_sh.at[pl.ds(s * PR, PR)])
            plsc.subcore_barrier()

            cnt = cv[ci]
            region = (((c * _NCH) + ci) * _NTILES + s) * _CAPB

            pltpu.async_copy(pk_hbm.at[pl.ds(region, _GRP)],
                             pkblk.at[pl.ds(0, _GRP)], sem_pk)

            def group(g, carry):
                base = (g & 1) * _GRP
                nbase = ((g + 1) & 1) * _GRP
                pltpu.make_async_copy(
                    pk_hbm.at[pl.ds(region + g * _GRP, _GRP)],
                    pkblk.at[pl.ds(base, _GRP)], sem_pk).wait()
                pltpu.async_copy(
                    pk_hbm.at[pl.ds(region + (g + 1) * _GRP, _GRP)],
                    pkblk.at[pl.ds(nbase, _GRP)], sem_pk)
                for q in range(_NQ):
                    for r in range(_B // _LANES):
                        pk = pkblk[pl.ds(base + q * _B + r * _LANES, _LANES)]
                        ex_src[q, pl.ds(r * _LANES, _LANES)] = pk & mask_src
                        ex_dst[q, pl.ds(r * _LANES, _LANES)] = \
                            lax.shift_right_logical(pk, jnp.int32(_SRC_BITS))
                for q in range(_NQ):
                    pltpu.async_copy(h_hbm.at[ex_src.at[q]], rows.at[q], sem_g)
                for q in range(_NQ):
                    pltpu.make_async_copy(h_hbm.at[ex_src.at[q]], rows.at[q],
                                          sem_g).wait()
                    pltpu.async_copy(rows.at[q], u_sh.at[ex_dst.at[q]], sem_s,
                                     add=True)
                for q in range(_NQ):
                    pltpu.make_async_copy(rows.at[q], u_sh.at[ex_dst.at[q]],
                                          sem_s).wait()
                return carry

            lax.fori_loop(0, cnt // _GRP + 1, group, jnp.int32(0))
            ng = cnt // _GRP + 1
            pltpu.make_async_copy(
                pk_hbm.at[pl.ds(region + ng * _GRP, _GRP)],
                pkblk.at[pl.ds((ng & 1) * _GRP, _GRP)], sem_pk).wait()
            plsc.subcore_barrier()

            pltpu.sync_copy(u_sh.at[pl.ds(s * PR, PR)],
                            u_hbm.at[pl.ds(lo + s * PR, PR)])
            plsc.subcore_barrier()

    return pl.kernel(
        body,
        mesh=plsc.VectorSubcoreMesh(core_axis_name="c", subcore_axis_name="s"),
        compiler_params=pltpu.CompilerParams(needs_layout_passes=False),
        out_type=jax.ShapeDtypeStruct((N_PAD, D), jnp.float32),
        scratch_types=[
            pltpu.VMEM((2 * _GRP,), jnp.int32),
            pltpu.VMEM((_LANES,), jnp.int32),
            pltpu.VMEM((_NQ, _B), jnp.int32),
            pltpu.VMEM((_NQ, _B), jnp.int32),
            pltpu.VMEM((_NQ, _B, D), jnp.float32),
            pltpu.SemaphoreType.DMA,
            pltpu.SemaphoreType.DMA,
            pltpu.SemaphoreType.DMA,
            pltpu.VMEM_SHARED((_C + 8, D), jnp.float32),
        ],
    )


_sc_agg_128 = _sc_aggregate_build(H_DIM)


def _aggregate(h, pk, cnts):
    return _sc_agg_128(h, pk, cnts)


def kernel(x, edge_index, batch, W1, b1, g1, bt1, W2, b2, g2, bt2,
           W3, b3, g3, bt3, Wfc, bfc):
    src = edge_index[0].astype(jnp.int32)
    dst = edge_index[1].astype(jnp.int32)
    batch = batch.astype(jnp.int32)

    pk, cnts = _sc_bucket(src, dst)
    y1 = _mm_plain(x, W1)
    u1 = _aggregate(y1, pk, cnts)
    s1 = _stats(u1)
    h1 = _norm(u1, s1, g1, bt1)

    u2 = _aggregate(h1, pk, cnts)
    z2, s2 = _mm_stats(u2, W2, b2)
    h2 = _norm(z2, s2, g2, bt2)

    u3 = _aggregate(h2, pk, cnts)
    z3, s3 = _mm_stats(u3, W3, b3)
    return _normpool(z3, s3, g3, bt3, batch, Wfc, bfc)
